# baseline (speedup 1.0000x reference)
import functools

import jax
import jax.numpy as jnp
from jax import lax
from jax.experimental import pallas as pl
from jax.experimental.pallas import tpu as pltpu
from jax.experimental.pallas import tpu_sc as plsc

_NC = 2
_NS = 16
_NW = _NC * _NS
_LANES = 16
_FP = 28
_DP = 64
_NBLK = _FP * _DP // 256
_FPC = 4


def _sc_gather(xc_flat, table_flat, V, BP):
    dt = table_flat.dtype
    BW = BP // _NW
    n_idx = BW * _FP
    n_vec = n_idx // _LANES
    CH = BW * _FPC
    n_ch = _FP // _FPC

    mesh = plsc.VectorSubcoreMesh(
        core_axis_name="c", subcore_axis_name="s",
        num_cores=_NC, num_subcores=_NS)

    @functools.partial(
        pl.kernel,
        out_type=jax.ShapeDtypeStruct((_NBLK, BP, 256), dt),
        mesh=mesh,
        scratch_types=[
            pltpu.VMEM((n_idx,), jnp.int32),
            pltpu.VMEM((CH, _DP), dt),
            pltpu.VMEM((CH, _DP), dt),
            pltpu.SemaphoreType.DMA,
            pltpu.SemaphoreType.DMA,
        ],
        compiler_params=pltpu.CompilerParams(use_tc_tiling_on_sc=False),
    )
    def k(xc_hbm, tab_hbm, out_hbm, idx_v, rows_a, rows_b, sem_a, sem_b):
        wid = lax.axis_index("s") * _NC + lax.axis_index("c")
        b0 = wid * BW
        pltpu.sync_copy(xc_hbm.at[pl.ds(wid * n_idx, n_idx)], idx_v)

        nv_per_f = BW // _LANES

        def idx_body(i, carry):
            f = i // nv_per_f
            sl = pl.ds(i * _LANES, _LANES)
            idx_v[sl] = idx_v[sl] + f * V
            return carry

        lax.fori_loop(0, n_vec, idx_body, 0, unroll=4)

        bufs = (rows_a, rows_b)
        sems = (sem_a, sem_b)

        def fire(c, buf, sem):
            return pltpu.async_copy(
                tab_hbm.at[idx_v.at[pl.ds(c * CH, CH)]], buf, sem)

        def drain(c, buf):
            for q in range(_FPC):
                f = c * _FPC + q
                pltpu.sync_copy(
                    buf.at[pl.ds(q * BW, BW)],
                    out_hbm.at[f // 4, pl.ds(b0, BW), pl.ds((f % 4) * _DP, _DP)])

        fire(0, bufs[0], sems[0]).wait()
        for c in range(n_ch):
            if c + 1 < n_ch:
                nxt = fire(c + 1, bufs[(c + 1) % 2], sems[(c + 1) % 2])
            drain(c, bufs[c % 2])
            if c + 1 < n_ch:
                nxt.wait()

    return k(xc_flat, table_flat)


def _stats_body(x_ref, g_ref, b_ref, o_ref):
    x = x_ref[...]
    n = x.shape[0]
    mu = jnp.sum(x, axis=0, keepdims=True) * (1.0 / n)
    var = jnp.sum(x * x, axis=0, keepdims=True) * (1.0 / n) - mu * mu
    s = g_ref[...] * lax.rsqrt(var + 1e-5)
    o_ref[0:1, :] = s
    o_ref[1:2, :] = b_ref[...] - mu * s


def _mlp_body(xnum_ref, sc_ref, embs_ref, w1lo_ref, w1hi_ref, w1b_ref,
              b1_ref, w2_ref, b2_ref, wo_ref, bo_ref, out_ref):
    xb = xnum_ref[...] * sc_ref[0:1, :] + sc_ref[1:2, :]
    h = jnp.dot(xb, w1b_ref[...], preferred_element_type=jnp.float32)
    mask = jnp.uint32(0xFFFF0000)
    for c in range(_NBLK):
        u = lax.bitcast_convert_type(embs_ref[c], jnp.uint32)
        lo = lax.bitcast_convert_type(u << 16, jnp.float32)
        hi = lax.bitcast_convert_type(u & mask, jnp.float32)
        h = h + jnp.dot(lo.astype(jnp.bfloat16), w1lo_ref[c],
                        preferred_element_type=jnp.float32)
        h = h + jnp.dot(hi.astype(jnp.bfloat16), w1hi_ref[c],
                        preferred_element_type=jnp.float32)
    h = jnp.maximum(h + b1_ref[...], 0.0)
    h2 = jnp.dot(h.astype(w2_ref.dtype), w2_ref[...],
                 preferred_element_type=jnp.float32)
    h2 = jnp.maximum(h2 + b2_ref[...], 0.0)
    o = jnp.dot(h2.astype(wo_ref.dtype), wo_ref[...],
                preferred_element_type=jnp.float32)
    out_ref[...] = jax.nn.sigmoid(o + bo_ref[...])


def kernel(x_cat, x_num, emb_tables, gamma, beta, W1, b1, W2, b2, Wout, bout):
    B, F = x_cat.shape
    _, V, D = emb_tables.shape
    NN = x_num.shape[1]
    H1 = W1.shape[1]
    H2 = W2.shape[1]
    ED = F * D

    NP = 16
    x_num_p = jnp.pad(x_num, ((0, 0), (0, NP - NN)))
    gamma_p = jnp.pad(gamma, (0, NP - NN)).reshape(1, NP)
    beta_p = jnp.pad(beta, (0, NP - NN)).reshape(1, NP)
    W1e = jnp.pad(W1[:ED].reshape(F, D, H1),
                  ((0, _FP - F), (0, _DP - D), (0, 0)))
    W1e = W1e.reshape(_NBLK, 128, 2, H1).astype(jnp.bfloat16)
    W1lo = W1e[:, :, 0, :]
    W1hi = W1e[:, :, 1, :]
    W1b = jnp.pad(W1[ED:], ((0, NP - NN), (0, 0)))
    W2b = W2.astype(jnp.bfloat16)
    Woutb = Wout.astype(jnp.bfloat16)
    tab = jnp.pad(emb_tables, ((0, _FP - F), (0, 0), (0, _DP - D)))
    tab = tab.astype(jnp.bfloat16).reshape(_FP * V, _DP)
    xc28 = jnp.pad(x_cat.astype(jnp.int32), ((0, 0), (0, _FP - F)))

    stats = pl.pallas_call(
        _stats_body,
        out_shape=jax.ShapeDtypeStruct((2, NP), jnp.float32),
    )(x_num_p, gamma_p, beta_p)

    NSPLIT = 2
    BP = B // NSPLIT
    BLK = 1024
    BW = BP // _NW
    xc_fm = xc28.reshape(NSPLIT * _NW, BW, _FP).transpose(0, 2, 1).reshape(-1)

    embs_parts = []
    for kpart in range(NSPLIT):
        o = _sc_gather(
            lax.slice_in_dim(xc_fm, kpart * BP * _FP, (kpart + 1) * BP * _FP),
            tab, V, BP)
        ow = lax.bitcast_convert_type(o.reshape(_NBLK, BP, 128, 2),
                                      jnp.float32)
        embs_parts.append(ow)

    mlp = pl.pallas_call(
        _mlp_body,
        grid=(BP // BLK,),
        in_specs=[
            pl.BlockSpec((BLK, NP), lambda i: (i, 0)),
            pl.BlockSpec((2, NP), lambda i: (0, 0)),
            pl.BlockSpec((_NBLK, BLK, 128), lambda i: (0, i, 0)),
            pl.BlockSpec((_NBLK, 128, H1), lambda i: (0, 0, 0)),
            pl.BlockSpec((_NBLK, 128, H1), lambda i: (0, 0, 0)),
            pl.BlockSpec((NP, H1), lambda i: (0, 0)),
            pl.BlockSpec((1, H1), lambda i: (0, 0)),
            pl.BlockSpec((H1, H2), lambda i: (0, 0)),
            pl.BlockSpec((1, H2), lambda i: (0, 0)),
            pl.BlockSpec((H2, 1), lambda i: (0, 0)),
            pl.BlockSpec((1, 1), lambda i: (0, 0)),
        ],
        out_specs=pl.BlockSpec((BLK, 1), lambda i: (i, 0)),
        out_shape=jax.ShapeDtypeStruct((BP, 1), jnp.float32),
    )
    outs = [
        mlp(lax.slice_in_dim(x_num_p, kpart * BP, (kpart + 1) * BP), stats,
            embs_parts[kpart], W1lo, W1hi, W1b, b1.reshape(1, H1), W2b,
            b2.reshape(1, H2), Woutb, bout.reshape(1, 1))
        for kpart in range(NSPLIT)
    ]
    return jnp.concatenate(outs, axis=0) if NSPLIT > 1 else outs[0]
